# baseline (speedup 1.0000x reference)
import jax
import jax.numpy as jnp
from jax import lax
from jax.experimental import pallas as pl
from jax.experimental.pallas import tpu as pltpu
from jax.experimental.pallas import tpu_sc as plsc

N = 320000
D = 128
S = 10000
CHUNK = 128
NC = 2
NS = 16
NCHUNKS = N // CHUNK
BASE_PER_W = NCHUNKS // NS
REM = NCHUNKS - BASE_PER_W * NS
PAIRS0 = (BASE_PER_W + 2) // 2
BLK = 4
NB = NCHUNKS // BLK
BASE_B = NB // NS
REM_B = NB - BASE_B * NS
PAIRS1 = (BASE_B + 2) // 2
NBLK = S // 128
BTAIL = S - NBLK * 128
BLK_ITERS = (NBLK + NS - 1) // NS
TAIL_SUB = NBLK - NS * (BLK_ITERS - 1)


def _sc_partials(features, seg_rows, zeros, ones):
    mesh = plsc.VectorSubcoreMesh(core_axis_name="c", subcore_axis_name="s")

    def body(feat_hbm, seg_hbm, zeros_hbm, ones_hbm, out_hbm,
             acc_s, rows_a, rows_b, idx_a, idx_b, blk_a, blk_b,
             sem_a, sem_b, scat_a, scat_b):
        c = lax.axis_index("c")
        s = lax.axis_index("s")

        pltpu.sync_copy(zeros_hbm, rows_a)
        for i in range(BLK_ITERS):
            b = s + NS * i

            @pl.when(b < NBLK)
            def _():
                pltpu.sync_copy(rows_a, acc_s.at[pl.ds(b * 128, 128)])

        @pl.when(s == TAIL_SUB)
        def _():
            pltpu.sync_copy(rows_a.at[pl.ds(0, BTAIL)],
                            acc_s.at[pl.ds(NBLK * 128, BTAIL)])

        plsc.subcore_barrier()

        @pl.when(c == 0)
        def _():
            nj = jnp.where(s < REM, BASE_PER_W + 1, BASE_PER_W)

            def load(k, idx_v, rows_v, sem):
                pltpu.async_copy(seg_hbm.at[k], idx_v, sem)
                pltpu.async_copy(feat_hbm.at[pl.ds(k * CHUNK, CHUNK)],
                                 rows_v, sem)

            def drain(k, idx_v, rows_v, sem):
                pltpu.make_async_copy(seg_hbm.at[k], idx_v, sem).wait()
                pltpu.make_async_copy(feat_hbm.at[pl.ds(k * CHUNK, CHUNK)],
                                      rows_v, sem).wait()

            load(s, idx_a, rows_a, sem_a)

            slots = ((idx_a, rows_a, sem_a, scat_a),
                     (idx_b, rows_b, sem_b, scat_b))

            def pair(jp, carry):
                for bi, (idx_v, rows_v, sem, scat) in enumerate(slots):
                    j = 2 * jp + bi
                    k = s + j * NS
                    idx_o, rows_o, sem_o, scat_o = slots[1 - bi]

                    @pl.when(j < nj)
                    def _():
                        drain(k, idx_v, rows_v, sem)
                        pltpu.async_copy(rows_v, acc_s.at[idx_v.at[0]],
                                         scat, add=True)

                        @pl.when(j >= 1)
                        def _():
                            pltpu.make_async_copy(
                                rows_o, acc_s.at[idx_o.at[0]], scat_o).wait()

                        @pl.when(j + 1 < nj)
                        def _():
                            load(k + NS, idx_o, rows_o, sem_o)

                return carry

            lax.fori_loop(0, PAIRS0, pair, 0)

            @pl.when(nj % 2 == 1)
            def _():
                pltpu.make_async_copy(rows_a, acc_s.at[idx_a.at[0]],
                                      scat_a).wait()

            @pl.when(nj % 2 == 0)
            def _():
                pltpu.make_async_copy(rows_b, acc_s.at[idx_b.at[0]],
                                      scat_b).wait()

        @pl.when(c == 1)
        def _():
            pltpu.sync_copy(ones_hbm, rows_a)
            nb = jnp.where(s < REM_B, BASE_B + 1, BASE_B)

            pltpu.async_copy(seg_hbm.at[pl.ds(s * BLK, BLK)], blk_a, sem_a)
            pltpu.async_copy(seg_hbm.at[pl.ds((s + NS) * BLK, BLK)],
                             blk_b, sem_b)

            def pair(jp, carry):
                for bi, (blk_v, sem) in enumerate(
                        ((blk_a, sem_a), (blk_b, sem_b))):
                    j = 2 * jp + bi
                    kb = s + j * NS

                    @pl.when(j < nb)
                    def _():
                        pltpu.make_async_copy(
                            seg_hbm.at[pl.ds(kb * BLK, BLK)], blk_v,
                            sem).wait()
                        for q in range(BLK):
                            pltpu.sync_copy(rows_a,
                                            acc_s.at[blk_v.at[q, 0]],
                                            add=True)

                        @pl.when(j + 2 < nb)
                        def _():
                            pltpu.async_copy(
                                seg_hbm.at[pl.ds((kb + 2 * NS) * BLK, BLK)],
                                blk_v, sem)

                return carry

            lax.fori_loop(0, PAIRS1, pair, 0)

        plsc.subcore_barrier()

        for i in range(BLK_ITERS):
            b = s + NS * i

            @pl.when(b < NBLK)
            def _():
                pltpu.sync_copy(acc_s.at[pl.ds(b * 128, 128)], rows_b)
                pltpu.sync_copy(rows_b, out_hbm.at[c, pl.ds(b * 128, 128)])

        @pl.when(s == TAIL_SUB)
        def _():
            pltpu.sync_copy(acc_s.at[pl.ds(NBLK * 128, BTAIL)],
                            rows_b.at[pl.ds(0, BTAIL)])
            pltpu.sync_copy(rows_b.at[pl.ds(0, BTAIL)],
                            out_hbm.at[c, pl.ds(NBLK * 128, BTAIL)])

    return pl.kernel(
        body,
        out_type=jax.ShapeDtypeStruct((NC, S, D), jnp.float32),
        mesh=mesh,
        scratch_types=[
            pltpu.VMEM_SHARED((S, D), jnp.float32),
            pltpu.VMEM((CHUNK, D), jnp.float32),
            pltpu.VMEM((CHUNK, D), jnp.float32),
            pltpu.VMEM((1, 128), jnp.int32),
            pltpu.VMEM((1, 128), jnp.int32),
            pltpu.VMEM((BLK, 1, 128), jnp.int32),
            pltpu.VMEM((BLK, 1, 128), jnp.int32),
            pltpu.SemaphoreType.DMA,
            pltpu.SemaphoreType.DMA,
            pltpu.SemaphoreType.DMA,
            pltpu.SemaphoreType.DMA,
        ],
    )(features, seg_rows, zeros, ones)


def _combine(parts):
    def body(p_ref, out_ref):
        sums = p_ref[0]
        cnt = p_ref[1, :, 0:1]
        out_ref[...] = jnp.where(cnt > 0.0, sums / jnp.maximum(cnt, 1.0), 0.0)

    rows = 1000
    return pl.pallas_call(
        body,
        grid=(S // rows,),
        in_specs=[pl.BlockSpec((NC, rows, D), lambda i: (0, i, 0))],
        out_specs=pl.BlockSpec((rows, D), lambda i: (i, 0)),
        out_shape=jax.ShapeDtypeStruct((S, D), jnp.float32),
    )(parts)


def kernel(features, segments):
    seg_rows = segments.reshape(NCHUNKS, CHUNK // 128, 128)
    zeros = jnp.zeros((128, D), jnp.float32)
    ones = jnp.ones((128, D), jnp.float32)
    parts = _sc_partials(features, seg_rows, zeros, ones)
    return _combine(parts)


# speedup vs baseline: 1.2060x; 1.2060x over previous
import jax
import jax.numpy as jnp
from jax import lax
from jax.experimental import pallas as pl
from jax.experimental.pallas import tpu as pltpu
from jax.experimental.pallas import tpu_sc as plsc

N = 320000
D = 128
S = 10000
CHUNK = 128
NC = 2
NS = 16
NCHUNKS = N // CHUNK
BASE_PER_W = NCHUNKS // NS
REM = NCHUNKS - BASE_PER_W * NS
PAIRS0 = (BASE_PER_W + 2) // 2
TRIPLES = (BASE_PER_W + 1 + 2) // 3
BLK = 4
NB = NCHUNKS // BLK
BASE_B = NB // NS
REM_B = NB - BASE_B * NS
PAIRS1 = (BASE_B + 2) // 2
NBLK = S // 128
BTAIL = S - NBLK * 128
BLK_ITERS = (NBLK + NS - 1) // NS
TAIL_SUB = NBLK - NS * (BLK_ITERS - 1)


def _sc_partials(features, seg_rows, zeros, ones):
    mesh = plsc.VectorSubcoreMesh(core_axis_name="c", subcore_axis_name="s")

    def body(feat_hbm, seg_hbm, zeros_hbm, ones_hbm, out_hbm,
             acc_s, rows_a, rows_b, rows_c, idx_a, idx_b, idx_c,
             sem_a, sem_b, sem_c, scat_a, scat_b, scat_c):
        c = lax.axis_index("c")
        s = lax.axis_index("s")

        pltpu.sync_copy(zeros_hbm, rows_a)
        for i in range(BLK_ITERS):
            b = s + NS * i

            @pl.when(b < NBLK)
            def _():
                pltpu.sync_copy(rows_a, acc_s.at[pl.ds(b * 128, 128)])

        @pl.when(s == TAIL_SUB)
        def _():
            pltpu.sync_copy(rows_a.at[pl.ds(0, BTAIL)],
                            acc_s.at[pl.ds(NBLK * 128, BTAIL)])

        plsc.subcore_barrier()

        @pl.when(c == 0)
        def _():
            nj = jnp.where(s < REM, BASE_PER_W + 1, BASE_PER_W)

            def load(k, idx_v, rows_v, sem):
                pltpu.async_copy(seg_hbm.at[k], idx_v, sem)
                pltpu.async_copy(feat_hbm.at[pl.ds(k * CHUNK, CHUNK)],
                                 rows_v, sem)

            def drain(k, idx_v, rows_v, sem):
                pltpu.make_async_copy(seg_hbm.at[k], idx_v, sem).wait()
                pltpu.make_async_copy(feat_hbm.at[pl.ds(k * CHUNK, CHUNK)],
                                      rows_v, sem).wait()

            slots = ((idx_a, rows_a, sem_a, scat_a),
                     (idx_b, rows_b, sem_b, scat_b),
                     (idx_c, rows_c, sem_c, scat_c))

            load(s, idx_a, rows_a, sem_a)
            load(s + NS, idx_b, rows_b, sem_b)

            def triple(jp, carry):
                for bi in range(3):
                    idx_v, rows_v, sem, scat = slots[bi]
                    idx_p, rows_p, sem_p, scat_p = slots[(bi + 2) % 3]
                    j = 3 * jp + bi
                    k = s + j * NS

                    @pl.when(j < nj)
                    def _():
                        drain(k, idx_v, rows_v, sem)

                        @pl.when(j >= 1)
                        def _():
                            pltpu.make_async_copy(
                                rows_p, acc_s.at[idx_p.at[0]], scat_p).wait()

                        pltpu.async_copy(rows_v, acc_s.at[idx_v.at[0]],
                                         scat, add=True)

                        @pl.when(j + 2 < nj)
                        def _():
                            load(k + 2 * NS, idx_p, rows_p, sem_p)

                return carry

            lax.fori_loop(0, TRIPLES, triple, 0)

            for t in range(3):
                idx_v, rows_v, sem, scat = slots[t]

                @pl.when((nj - 1) % 3 == t)
                def _():
                    pltpu.make_async_copy(rows_v, acc_s.at[idx_v.at[0]],
                                          scat).wait()

        @pl.when(c == 1)
        def _():
            pltpu.sync_copy(ones_hbm, rows_a)
            nb = jnp.where(s < REM, BASE_PER_W + 1, BASE_PER_W)

            pltpu.async_copy(seg_hbm.at[s], idx_a, sem_a)
            pltpu.async_copy(seg_hbm.at[s + NS], idx_b, sem_b)

            def pair(jp, carry):
                for bi, (idx_v, sem) in enumerate(
                        ((idx_a, sem_a), (idx_b, sem_b))):
                    j = 2 * jp + bi
                    k = s + j * NS

                    @pl.when(j < nb)
                    def _():
                        pltpu.make_async_copy(seg_hbm.at[k], idx_v,
                                              sem).wait()
                        pltpu.sync_copy(rows_a, acc_s.at[idx_v.at[0]],
                                        add=True)

                        @pl.when(j + 2 < nb)
                        def _():
                            pltpu.async_copy(seg_hbm.at[k + 2 * NS],
                                             idx_v, sem)

                return carry

            lax.fori_loop(0, PAIRS0, pair, 0)

        plsc.subcore_barrier()

        for i in range(BLK_ITERS):
            b = s + NS * i

            @pl.when(b < NBLK)
            def _():
                pltpu.sync_copy(acc_s.at[pl.ds(b * 128, 128)], rows_b)
                pltpu.sync_copy(rows_b, out_hbm.at[c, pl.ds(b * 128, 128)])

        @pl.when(s == TAIL_SUB)
        def _():
            pltpu.sync_copy(acc_s.at[pl.ds(NBLK * 128, BTAIL)],
                            rows_b.at[pl.ds(0, BTAIL)])
            pltpu.sync_copy(rows_b.at[pl.ds(0, BTAIL)],
                            out_hbm.at[c, pl.ds(NBLK * 128, BTAIL)])

    return pl.kernel(
        body,
        out_type=jax.ShapeDtypeStruct((NC, S, D), jnp.float32),
        mesh=mesh,
        scratch_types=[
            pltpu.VMEM_SHARED((S, D), jnp.float32),
            pltpu.VMEM((CHUNK, D), jnp.float32),
            pltpu.VMEM((CHUNK, D), jnp.float32),
            pltpu.VMEM((CHUNK, D), jnp.float32),
            pltpu.VMEM((1, 128), jnp.int32),
            pltpu.VMEM((1, 128), jnp.int32),
            pltpu.VMEM((1, 128), jnp.int32),
            pltpu.SemaphoreType.DMA,
            pltpu.SemaphoreType.DMA,
            pltpu.SemaphoreType.DMA,
            pltpu.SemaphoreType.DMA,
            pltpu.SemaphoreType.DMA,
            pltpu.SemaphoreType.DMA,
        ],
    )(features, seg_rows, zeros, ones)


def _combine(parts):
    def body(p_ref, out_ref):
        sums = p_ref[0]
        cnt = p_ref[1, :, 0:1]
        out_ref[...] = jnp.where(cnt > 0.0, sums / jnp.maximum(cnt, 1.0), 0.0)

    rows = 1000
    return pl.pallas_call(
        body,
        grid=(S // rows,),
        in_specs=[pl.BlockSpec((NC, rows, D), lambda i: (0, i, 0))],
        out_specs=pl.BlockSpec((rows, D), lambda i: (i, 0)),
        out_shape=jax.ShapeDtypeStruct((S, D), jnp.float32),
    )(parts)


def kernel(features, segments):
    seg_rows = segments.reshape(NCHUNKS, CHUNK // 128, 128)
    zeros = jnp.zeros((128, D), jnp.float32)
    ones = jnp.ones((128, D), jnp.float32)
    parts = _sc_partials(features, seg_rows, zeros, ones)
    return _combine(parts)
